# baseline (speedup 1.0000x reference)
import functools

import jax
import jax.numpy as jnp
from jax import lax
from jax.experimental import pallas as pl
from jax.experimental.pallas import tpu as pltpu
from jax.experimental.pallas import tpu_sc as plsc

D_MODEL = 128
CHUNK = 64
NBUF = 12
LOOKAHEAD = NBUF // 2
NUM_CORES = 2
NUM_SUBCORES = 16
NUM_WORKERS = NUM_CORES * NUM_SUBCORES


@functools.partial(jax.jit, static_argnums=(2, 3))
def _gather_rows(idx3, table, n_rows, n_chunks):
    rows_per_w = n_chunks * CHUNK
    mesh = plsc.VectorSubcoreMesh(core_axis_name="c", subcore_axis_name="s")

    @functools.partial(
        pl.kernel,
        out_type=jax.ShapeDtypeStruct((n_rows, D_MODEL), jnp.float32),
        mesh=mesh,
        scratch_types=[
            pltpu.VMEM((n_chunks, CHUNK), jnp.int32),
            [pltpu.VMEM((CHUNK, D_MODEL), jnp.float32) for _ in range(NBUF)],
            [pltpu.SemaphoreType.DMA for _ in range(NBUF)],
            [pltpu.SemaphoreType.DMA for _ in range(NBUF)],
        ],
    )
    def k(idx_hbm, table_hbm, out_hbm, idx_v, rows, gsem, wsem):
        wid = lax.axis_index("s") * NUM_CORES + lax.axis_index("c")
        base = wid * rows_per_w
        pltpu.sync_copy(idx_hbm.at[wid], idx_v)

        def issue_gather(c, b):
            pltpu.async_copy(table_hbm.at[idx_v.at[c]], rows[b], gsem[b])

        def wait_gather(b):
            pltpu.make_async_copy(
                table_hbm.at[pl.ds(0, CHUNK)], rows[b], gsem[b]).wait()

        def issue_write(c, b):
            pltpu.async_copy(
                rows[b], out_hbm.at[pl.ds(base + c * CHUNK, CHUNK)], wsem[b])

        def wait_write(b):
            pltpu.make_async_copy(
                rows[b], out_hbm.at[pl.ds(base, CHUNK)], wsem[b]).wait()

        L = LOOKAHEAD
        for c in range(L):
            issue_gather(c, c)
        for c in range(L):
            issue_gather(c + L, (c + L) % NBUF)
            wait_gather(c % NBUF)
            issue_write(c, c % NBUF)

        mid = n_chunks - 2 * L
        mid_loop = (mid // NBUF) * NBUF

        def steady_step(c, b):
            b2 = (b + L) % NBUF
            wait_write(b2)
            issue_gather(c + L, b2)
            wait_gather(b)
            issue_write(c, b)

        @pl.loop(L, L + mid_loop, step=NBUF)
        def _(j):
            for u in range(NBUF):
                steady_step(j + u, (L + u) % NBUF)

        for c in range(L + mid_loop, n_chunks - L):
            steady_step(c, c % NBUF)

        for c in range(n_chunks - L, n_chunks):
            wait_write((c + L) % NBUF)
            wait_gather(c % NBUF)
            issue_write(c, c % NBUF)
        for c in range(n_chunks - L, n_chunks):
            wait_write(c % NBUF)

    return k(idx3, table)


def kernel(enc_inputs, emb_table):
    batch, seq = enc_inputs.shape
    n_rows = batch * seq
    n_chunks = n_rows // (NUM_WORKERS * CHUNK)
    idx3 = enc_inputs.reshape(NUM_WORKERS, n_chunks, CHUNK)
    out = _gather_rows(idx3, emb_table, n_rows, n_chunks)
    return out.reshape(batch, seq, D_MODEL)


# speedup vs baseline: 1.0114x; 1.0114x over previous
import functools

import jax
import jax.numpy as jnp
from jax import lax
from jax.experimental import pallas as pl
from jax.experimental.pallas import tpu as pltpu
from jax.experimental.pallas import tpu_sc as plsc

D_MODEL = 128
CHUNK = 80
NBUF = 10
LOOKAHEAD = NBUF // 2
NUM_CORES = 2
NUM_SUBCORES = 16
NUM_WORKERS = NUM_CORES * NUM_SUBCORES


@functools.partial(jax.jit, static_argnums=(2, 3))
def _gather_rows(idx3, table, n_rows, n_chunks):
    rows_per_w = n_chunks * CHUNK
    mesh = plsc.VectorSubcoreMesh(core_axis_name="c", subcore_axis_name="s")

    @functools.partial(
        pl.kernel,
        out_type=jax.ShapeDtypeStruct((n_rows, D_MODEL), jnp.float32),
        mesh=mesh,
        scratch_types=[
            pltpu.VMEM((n_chunks, CHUNK), jnp.int32),
            [pltpu.VMEM((CHUNK, D_MODEL), jnp.float32) for _ in range(NBUF)],
            [pltpu.SemaphoreType.DMA for _ in range(NBUF)],
            [pltpu.SemaphoreType.DMA for _ in range(NBUF)],
        ],
    )
    def k(idx_hbm, table_hbm, out_hbm, idx_v, rows, gsem, wsem):
        wid = lax.axis_index("s") * NUM_CORES + lax.axis_index("c")
        base = wid * rows_per_w
        pltpu.sync_copy(idx_hbm.at[wid], idx_v)

        def issue_gather(c, b):
            pltpu.async_copy(table_hbm.at[idx_v.at[c]], rows[b], gsem[b])

        def wait_gather(b):
            pltpu.make_async_copy(
                table_hbm.at[pl.ds(0, CHUNK)], rows[b], gsem[b]).wait()

        def issue_write(c, b):
            pltpu.async_copy(
                rows[b], out_hbm.at[pl.ds(base + c * CHUNK, CHUNK)], wsem[b])

        def wait_write(b):
            pltpu.make_async_copy(
                rows[b], out_hbm.at[pl.ds(base, CHUNK)], wsem[b]).wait()

        L = LOOKAHEAD
        for c in range(L):
            issue_gather(c, c)
        for c in range(L):
            issue_gather(c + L, (c + L) % NBUF)
            wait_gather(c % NBUF)
            issue_write(c, c % NBUF)

        mid = n_chunks - 2 * L
        mid_loop = (mid // NBUF) * NBUF

        def steady_step(c, b):
            b2 = (b + L) % NBUF
            wait_write(b2)
            issue_gather(c + L, b2)
            wait_gather(b)
            issue_write(c, b)

        @pl.loop(L, L + mid_loop, step=NBUF)
        def _(j):
            for u in range(NBUF):
                steady_step(j + u, (L + u) % NBUF)

        for c in range(L + mid_loop, n_chunks - L):
            steady_step(c, c % NBUF)

        for c in range(n_chunks - L, n_chunks):
            wait_write((c + L) % NBUF)
            wait_gather(c % NBUF)
            issue_write(c, c % NBUF)
        for c in range(n_chunks - L, n_chunks):
            wait_write(c % NBUF)

    return k(idx3, table)


def kernel(enc_inputs, emb_table):
    batch, seq = enc_inputs.shape
    n_rows = batch * seq
    n_chunks = n_rows // (NUM_WORKERS * CHUNK)
    idx3 = enc_inputs.reshape(NUM_WORKERS, n_chunks, CHUNK)
    out = _gather_rows(idx3, emb_table, n_rows, n_chunks)
    return out.reshape(batch, seq, D_MODEL)
